# Initial kernel scaffold; baseline (speedup 1.0000x reference)
import jax
import jax.numpy as jnp
from jax.experimental import pallas as pl


def kernel(problems, encoded_nodes, prefix, Wq_first, Wq_last, Wk, Wv, Wo):
    raise NotImplementedError("write your pallas kernel here")


# speedup vs baseline: 1.0508x; 1.0508x over previous
import functools

import jax
import jax.numpy as jnp
from jax import lax
from jax.experimental import pallas as pl
from jax.experimental.pallas import tpu as pltpu

H = 8
CLIP = 10.0
NEG = -1e9


def _decode_body(enc_ref, ninf_ref, last_ref, wqf_ref, wql_ref, wk_ref,
                 wv_ref, wo_ref, sels_ref, *, n_steps):
    cb, p, d = enc_ref.shape[0], enc_ref.shape[1], enc_ref.shape[2]
    dh = d // H
    f32 = jnp.float32
    hi = jax.lax.Precision.HIGHEST
    df = jax.lax.Precision.DEFAULT

    enc = enc_ref[...]
    wk = wk_ref[...]
    wv = wv_ref[...]
    k = lax.dot_general(enc, wk, (((2,), (0,)), ((), ())),
                        precision=df, preferred_element_type=f32)
    v = lax.dot_general(enc, wv, (((2,), (0,)), ((), ())),
                        precision=df, preferred_element_type=f32)

    iota_p = lax.broadcasted_iota(jnp.int32, (cb, p), 1)
    gm = (lax.broadcasted_iota(jnp.int32, (H, d), 1) // dh
          == lax.broadcasted_iota(jnp.int32, (H, d), 0)).astype(f32)

    lastv = last_ref[:, 0, 0:1]
    onehot = (iota_p == lastv).astype(f32)
    ninf = ninf_ref[:, 0, :]

    g0 = lax.dot_general(onehot[:, None, :], enc, (((2,), (1,)), ((0,), (0,))),
                         precision=hi, preferred_element_type=f32)[:, 0, :]
    qf = jnp.dot(g0, wqf_ref[...], precision=df, preferred_element_type=f32)

    sels_ref[...] = jnp.zeros(sels_ref.shape, jnp.int32)

    for t in range(n_steps):
        g = lax.dot_general(onehot[:, None, :], enc,
                            (((2,), (1,)), ((0,), (0,))),
                            precision=hi, preferred_element_type=f32)[:, 0, :]
        q = qf + jnp.dot(g, wql_ref[...], precision=df,
                         preferred_element_type=f32)
        qh = (q[:, None, :] * gm[None]) * (1.0 / (dh ** 0.5))
        logits = lax.dot_general(qh, k, (((2,), (2,)), ((0,), (0,))),
                                 precision=df, preferred_element_type=f32)
        logits = logits + ninf[:, None, :]
        m = jnp.max(logits, axis=2, keepdims=True)
        e = jnp.exp(logits - m)
        aw = e / jnp.sum(e, axis=2, keepdims=True)
        mh_g = lax.dot_general(aw, v, (((2,), (1,)), ((0,), (0,))),
                               precision=df, preferred_element_type=f32)
        mh = jnp.sum(mh_g * gm[None], axis=1)
        mo = jnp.dot(mh, wo_ref[...], precision=df, preferred_element_type=f32)
        sc = lax.dot_general(mo[:, None, :], enc, (((2,), (2,)), ((0,), (0,))),
                             precision=df, preferred_element_type=f32)[:, 0, :]
        sc = CLIP * jnp.tanh(sc * (1.0 / (d ** 0.5))) + ninf
        m2 = jnp.max(sc, axis=1, keepdims=True)
        idx = jnp.min(jnp.where(sc == m2, iota_p, p), axis=1, keepdims=True)
        sels_ref[:, 0, t:t + 1] = idx
        onehot = (iota_p == idx).astype(f32)
        ninf = jnp.where(iota_p == idx, NEG, ninf)


def _decode(encoded_nodes, ninf0, last2d, Wq_first, Wq_last, Wk, Wv, Wo,
            n_steps, cb):
    b, p, d = encoded_nodes.shape
    grid = b // cb
    wspec = pl.BlockSpec((d, d), lambda i: (0, 0))
    return pl.pallas_call(
        functools.partial(_decode_body, n_steps=n_steps),
        grid=(grid,),
        in_specs=[
            pl.BlockSpec((cb, p, d), lambda i: (i, 0, 0)),
            pl.BlockSpec((cb, 1, p), lambda i: (i, 0, 0)),
            pl.BlockSpec((cb, 1, 128), lambda i: (i, 0, 0)),
            wspec, wspec, wspec, wspec, wspec,
        ],
        out_specs=pl.BlockSpec((cb, 1, 128), lambda i: (i, 0, 0)),
        out_shape=jax.ShapeDtypeStruct((b, 1, 128), jnp.int32),
    )(encoded_nodes, ninf0[:, None, :], last2d[:, None, :],
      Wq_first, Wq_last, Wk, Wv, Wo)


def kernel(problems, encoded_nodes, prefix, Wq_first, Wq_last, Wk, Wv, Wo):
    b, p, d = encoded_nodes.shape
    kp = prefix.shape[1]
    n_steps = p - kp

    visited = jnp.zeros((b, p), dtype=bool)
    visited = visited.at[jnp.arange(b)[:, None], prefix].set(True)
    ninf0 = jnp.where(visited, NEG, 0.0).astype(jnp.float32)

    last2d = jnp.broadcast_to(prefix[:, -1:], (b, 128))
    sels = _decode(encoded_nodes, ninf0, last2d, Wq_first, Wq_last, Wk, Wv, Wo,
                   n_steps, cb=4)
    return jnp.concatenate([prefix, sels[:, 0, :n_steps]], axis=1)


# speedup vs baseline: 2.2542x; 2.1451x over previous
import functools

import jax
import jax.numpy as jnp
from jax import lax
from jax.experimental import pallas as pl
from jax.experimental.pallas import tpu as pltpu
from jax.experimental.pallas import tpu_sc as plsc

H = 8
CLIP = 10.0
NEG = -1e9


def _mask_body(prefix_hbm, out_hbm, idx_v, row_v, *, rows_per, p, kp, nc):
    wid = lax.axis_index("s") * nc + lax.axis_index("c")
    for r in range(rows_per):
        row = wid * rows_per + r
        pltpu.sync_copy(prefix_hbm.at[row], idx_v)

        for j in range(p // 16):
            row_v[pl.ds(j * 16, 16)] = jnp.zeros((16,), jnp.float32)
        for j in range(kp // 16):
            idx = idx_v[pl.ds(j * 16, 16)]
            plsc.store_scatter(row_v, [idx], jnp.full((16,), NEG, jnp.float32))
        if kp % 16:
            idx = idx_v[pl.ds(kp - 16, 16)]
            plsc.store_scatter(row_v, [idx], jnp.full((16,), NEG, jnp.float32))
        pltpu.sync_copy(row_v, out_hbm.at[row])


def _sc_mask(prefix, p):
    b, kp = prefix.shape
    info = plsc.get_sparse_core_info()
    nc, ns = info.num_cores, info.num_subcores
    nw = nc * ns
    import functools as _ft
    fn = pl.kernel(
        _ft.partial(_mask_body, rows_per=b // nw, p=p, kp=kp, nc=nc),
        out_type=jax.ShapeDtypeStruct((b, p), jnp.float32),
        mesh=plsc.VectorSubcoreMesh(core_axis_name="c", subcore_axis_name="s"),
        compiler_params=pltpu.CompilerParams(needs_layout_passes=False),
        scratch_types=[
            pltpu.VMEM((kp,), jnp.int32),
            pltpu.VMEM((p,), jnp.float32),
        ],
    )
    return fn(prefix)


def _decode_body(enc_ref, ninf_ref, last_ref, wqf_ref, wql_ref, wk_ref,
                 wv_ref, wo_ref, sels_ref, *, n_steps):
    cb, p, d = enc_ref.shape[0], enc_ref.shape[1], enc_ref.shape[2]
    dh = d // H
    f32 = jnp.float32
    hi = jax.lax.Precision.HIGHEST
    df = jax.lax.Precision.DEFAULT

    enc = enc_ref[...]
    wk = wk_ref[...]
    wv = wv_ref[...]
    k = lax.dot_general(enc, wk, (((2,), (0,)), ((), ())),
                        precision=df, preferred_element_type=f32)
    v = lax.dot_general(enc, wv, (((2,), (0,)), ((), ())),
                        precision=df, preferred_element_type=f32)

    iota_p = lax.broadcasted_iota(jnp.int32, (cb, p), 1)
    gm = (lax.broadcasted_iota(jnp.int32, (H, d), 1) // dh
          == lax.broadcasted_iota(jnp.int32, (H, d), 0)).astype(f32)

    lastv = last_ref[:, 0, 0:1]
    onehot = (iota_p == lastv).astype(f32)
    ninf = ninf_ref[:, 0, :]

    g0 = lax.dot_general(onehot[:, None, :], enc, (((2,), (1,)), ((0,), (0,))),
                         precision=hi, preferred_element_type=f32)[:, 0, :]
    qf = jnp.dot(g0, wqf_ref[...], precision=df, preferred_element_type=f32)

    sels_ref[...] = jnp.zeros(sels_ref.shape, jnp.int32)

    for t in range(n_steps):
        g = lax.dot_general(onehot[:, None, :], enc,
                            (((2,), (1,)), ((0,), (0,))),
                            precision=hi, preferred_element_type=f32)[:, 0, :]
        q = qf + jnp.dot(g, wql_ref[...], precision=df,
                         preferred_element_type=f32)
        qh = (q[:, None, :] * gm[None]) * (1.0 / (dh ** 0.5))
        logits = lax.dot_general(qh, k, (((2,), (2,)), ((0,), (0,))),
                                 precision=df, preferred_element_type=f32)
        logits = logits + ninf[:, None, :]
        m = jnp.max(logits, axis=2, keepdims=True)
        e = jnp.exp(logits - m)
        aw = e / jnp.sum(e, axis=2, keepdims=True)
        mh_g = lax.dot_general(aw, v, (((2,), (1,)), ((0,), (0,))),
                               precision=df, preferred_element_type=f32)
        mh = jnp.sum(mh_g * gm[None], axis=1)
        mo = jnp.dot(mh, wo_ref[...], precision=df, preferred_element_type=f32)
        sc = lax.dot_general(mo[:, None, :], enc, (((2,), (2,)), ((0,), (0,))),
                             precision=df, preferred_element_type=f32)[:, 0, :]
        sc = CLIP * jnp.tanh(sc * (1.0 / (d ** 0.5))) + ninf
        m2 = jnp.max(sc, axis=1, keepdims=True)
        idx = jnp.min(jnp.where(sc == m2, iota_p, p), axis=1, keepdims=True)
        sels_ref[:, 0, t:t + 1] = idx
        onehot = (iota_p == idx).astype(f32)
        ninf = jnp.where(iota_p == idx, NEG, ninf)


def _decode(encoded_nodes, ninf0, last2d, Wq_first, Wq_last, Wk, Wv, Wo,
            n_steps, cb):
    b, p, d = encoded_nodes.shape
    grid = b // cb
    wspec = pl.BlockSpec((d, d), lambda i: (0, 0))
    return pl.pallas_call(
        functools.partial(_decode_body, n_steps=n_steps),
        grid=(grid,),
        in_specs=[
            pl.BlockSpec((cb, p, d), lambda i: (i, 0, 0)),
            pl.BlockSpec((cb, 1, p), lambda i: (i, 0, 0)),
            pl.BlockSpec((cb, 1, 128), lambda i: (i, 0, 0)),
            wspec, wspec, wspec, wspec, wspec,
        ],
        out_specs=pl.BlockSpec((cb, 1, 128), lambda i: (i, 0, 0)),
        out_shape=jax.ShapeDtypeStruct((b, 1, 128), jnp.int32),
    )(encoded_nodes, ninf0[:, None, :], last2d[:, None, :],
      Wq_first, Wq_last, Wk, Wv, Wo)


def kernel(problems, encoded_nodes, prefix, Wq_first, Wq_last, Wk, Wv, Wo):
    b, p, d = encoded_nodes.shape
    kp = prefix.shape[1]
    n_steps = p - kp

    ninf0 = _sc_mask(prefix, p)

    last2d = jnp.broadcast_to(prefix[:, -1:], (b, 128))
    sels = _decode(encoded_nodes, ninf0, last2d, Wq_first, Wq_last, Wk, Wv, Wo,
                   n_steps, cb=4)
    return jnp.concatenate([prefix, sels[:, 0, :n_steps]], axis=1)


# speedup vs baseline: 4.6523x; 2.0638x over previous
import functools

import jax
import jax.numpy as jnp
from jax import lax
from jax.experimental import pallas as pl
from jax.experimental.pallas import tpu as pltpu
from jax.experimental.pallas import tpu_sc as plsc

H = 8
CLIP = 10.0
NEG = -1e9


def _mask_body(prefix_hbm, out_hbm, idx_v, row_v, *, rows_per, p, kp, nc):
    wid = lax.axis_index("s") * nc + lax.axis_index("c")
    for r in range(rows_per):
        row = wid * rows_per + r
        pltpu.sync_copy(prefix_hbm.at[row], idx_v)

        for j in range(p // 16):
            row_v[pl.ds(j * 16, 16)] = jnp.zeros((16,), jnp.float32)
        for j in range(kp // 16):
            idx = idx_v[pl.ds(j * 16, 16)]
            plsc.store_scatter(row_v, [idx], jnp.full((16,), NEG, jnp.float32))
        if kp % 16:
            idx = idx_v[pl.ds(kp - 16, 16)]
            plsc.store_scatter(row_v, [idx], jnp.full((16,), NEG, jnp.float32))
        pltpu.sync_copy(row_v, out_hbm.at[row])


def _sc_mask(prefix, p):
    b, kp = prefix.shape
    info = plsc.get_sparse_core_info()
    nc, ns = info.num_cores, info.num_subcores
    nw = nc * ns
    import functools as _ft
    fn = pl.kernel(
        _ft.partial(_mask_body, rows_per=b // nw, p=p, kp=kp, nc=nc),
        out_type=jax.ShapeDtypeStruct((b, p), jnp.float32),
        mesh=plsc.VectorSubcoreMesh(core_axis_name="c", subcore_axis_name="s"),
        compiler_params=pltpu.CompilerParams(needs_layout_passes=False),
        scratch_types=[
            pltpu.VMEM((kp,), jnp.int32),
            pltpu.VMEM((p,), jnp.float32),
        ],
    )
    return fn(prefix)


def _decode_body(enc_ref, ninf_ref, last_ref, wqf_ref, wql_ref, wk_ref,
                 wv_ref, wo_ref, sels_ref, *, n_steps, groups):
    cb, p, d = enc_ref.shape[0], enc_ref.shape[1], enc_ref.shape[2]
    dh = d // H
    gs = cb // groups
    f32 = jnp.float32
    df = jax.lax.Precision.DEFAULT

    iota_p = lax.broadcasted_iota(jnp.int32, (gs, p), 1)
    gm = (lax.broadcasted_iota(jnp.int32, (H, d), 1) // dh
          == lax.broadcasted_iota(jnp.int32, (H, d), 0)).astype(f32)

    def gather_rows(scalars, base):
        rows = [enc_ref[pl.ds(base + b, 1), pl.ds(scalars[b], 1), :]
                for b in range(gs)]
        return jnp.concatenate(rows, axis=0)[:, 0, :]

    sels_ref[...] = jnp.zeros(sels_ref.shape, jnp.int32)

    st = []
    for gi in range(groups):
        base = gi * gs
        enc_g = enc_ref[pl.ds(base, gs), :, :]
        k_g = lax.dot_general(enc_g, wk_ref[...], (((2,), (0,)), ((), ())),
                              precision=df, preferred_element_type=f32)
        v_g = lax.dot_general(enc_g, wv_ref[...], (((2,), (0,)), ((), ())),
                              precision=df, preferred_element_type=f32)
        cur = [last_ref[0, 0, base + b] for b in range(gs)]
        g0 = gather_rows(cur, base)
        qf = jnp.dot(g0, wqf_ref[...], precision=df,
                     preferred_element_type=f32)
        ninf_g = ninf_ref[pl.ds(base, gs), 0, :]
        st.append(dict(base=base, enc=enc_g, k=k_g, v=v_g, qf=qf,
                       cur=cur, ninf=ninf_g))

    for t in range(n_steps):
        for s in st:
            g = gather_rows(s["cur"], s["base"])
            q = s["qf"] + jnp.dot(g, wql_ref[...], precision=df,
                                  preferred_element_type=f32)
            qh = (q[:, None, :] * gm[None]) * (1.0 / (dh ** 0.5))
            logits = lax.dot_general(qh, s["k"], (((2,), (2,)), ((0,), (0,))),
                                     precision=df, preferred_element_type=f32)
            logits = logits + s["ninf"][:, None, :]
            m = jnp.max(logits, axis=2, keepdims=True)
            e = jnp.exp(logits - m)
            aw = e / jnp.sum(e, axis=2, keepdims=True)
            mh_g = lax.dot_general(aw, s["v"], (((2,), (1,)), ((0,), (0,))),
                                   precision=df, preferred_element_type=f32)
            mh = jnp.sum(mh_g * gm[None], axis=1)
            mo = jnp.dot(mh, wo_ref[...], precision=df,
                         preferred_element_type=f32)
            sc = lax.dot_general(mo[:, None, :], s["enc"],
                                 (((2,), (2,)), ((0,), (0,))),
                                 precision=df,
                                 preferred_element_type=f32)[:, 0, :]
            sc = sc + s["ninf"]
            m2 = jnp.max(sc, axis=1, keepdims=True)
            idx = jnp.min(jnp.where(sc == m2, iota_p, p), axis=1,
                          keepdims=True)
            sels_ref[pl.ds(s["base"], gs), 0, t:t + 1] = idx
            s["ninf"] = jnp.where(iota_p == idx, NEG, s["ninf"])
            s["cur"] = [jnp.min(idx[b]) for b in range(gs)]


def _decode(encoded_nodes, ninf0, last2d, Wq_first, Wq_last, Wk, Wv, Wo,
            n_steps, cb):
    b, p, d = encoded_nodes.shape
    grid = b // cb
    wspec = pl.BlockSpec((d, d), lambda i: (0, 0))
    return pl.pallas_call(
        functools.partial(_decode_body, n_steps=n_steps, groups=1),
        grid=(grid,),
        in_specs=[
            pl.BlockSpec((cb, p, d), lambda i: (i, 0, 0)),
            pl.BlockSpec((cb, 1, p), lambda i: (i, 0, 0)),
            pl.BlockSpec((1, 1, cb), lambda i: (i, 0, 0),
                         memory_space=pltpu.SMEM),
            wspec, wspec, wspec, wspec, wspec,
        ],
        out_specs=pl.BlockSpec((cb, 1, 128), lambda i: (i, 0, 0)),
        out_shape=jax.ShapeDtypeStruct((b, 1, 128), jnp.int32),
    )(encoded_nodes, ninf0[:, None, :], last2d.reshape(grid, 1, cb),
      Wq_first, Wq_last, Wk, Wv, Wo)


def kernel(problems, encoded_nodes, prefix, Wq_first, Wq_last, Wk, Wv, Wo):
    b, p, d = encoded_nodes.shape
    kp = prefix.shape[1]
    n_steps = p - kp

    ninf0 = _sc_mask(prefix, p)

    last2d = prefix[:, -1]
    sels = _decode(encoded_nodes, ninf0, last2d, Wq_first, Wq_last, Wk, Wv, Wo,
                   n_steps, cb=8)
    return jnp.concatenate([prefix, sels[:, 0, :n_steps]], axis=1)


# speedup vs baseline: 5.5162x; 1.1857x over previous
import functools

import jax
import jax.numpy as jnp
from jax import lax
from jax.experimental import pallas as pl
from jax.experimental.pallas import tpu as pltpu
from jax.experimental.pallas import tpu_sc as plsc

H = 8
CLIP = 10.0
NEG = -1e9


def _mask_body(prefix_hbm, out_hbm, idx_v, row_v, *, rows_per, p, kp, nc):
    wid = lax.axis_index("s") * nc + lax.axis_index("c")
    for r in range(rows_per):
        row = wid * rows_per + r
        pltpu.sync_copy(prefix_hbm.at[row], idx_v)

        for j in range(p // 16):
            row_v[pl.ds(j * 16, 16)] = jnp.zeros((16,), jnp.float32)
        for j in range(kp // 16):
            idx = idx_v[pl.ds(j * 16, 16)]
            plsc.store_scatter(row_v, [idx], jnp.full((16,), NEG, jnp.float32))
        if kp % 16:
            idx = idx_v[pl.ds(kp - 16, 16)]
            plsc.store_scatter(row_v, [idx], jnp.full((16,), NEG, jnp.float32))
        pltpu.sync_copy(row_v, out_hbm.at[row])


def _sc_mask(prefix, p):
    b, kp = prefix.shape
    info = plsc.get_sparse_core_info()
    nc, ns = info.num_cores, info.num_subcores
    nw = nc * ns
    import functools as _ft
    fn = pl.kernel(
        _ft.partial(_mask_body, rows_per=b // nw, p=p, kp=kp, nc=nc),
        out_type=jax.ShapeDtypeStruct((b, p), jnp.float32),
        mesh=plsc.VectorSubcoreMesh(core_axis_name="c", subcore_axis_name="s"),
        compiler_params=pltpu.CompilerParams(needs_layout_passes=False),
        scratch_types=[
            pltpu.VMEM((kp,), jnp.int32),
            pltpu.VMEM((p,), jnp.float32),
        ],
    )
    return fn(prefix)


def _decode_body(enc_ref, ninf_ref, last_ref, wqf_ref, wql_ref, wk_ref,
                 wv_ref, wo_ref, sels_ref, *, n_steps, groups):
    cb, p, d = enc_ref.shape[0], enc_ref.shape[1], enc_ref.shape[2]
    dh = d // H
    gs = cb // groups
    f32 = jnp.float32
    df = jax.lax.Precision.DEFAULT

    iota_p = lax.broadcasted_iota(jnp.int32, (gs, p), 1)
    eye_g = (lax.broadcasted_iota(jnp.int32, (gs, gs), 0)
             == lax.broadcasted_iota(jnp.int32, (gs, gs), 1)).astype(f32)
    gm = (lax.broadcasted_iota(jnp.int32, (H, d), 1) // dh
          == lax.broadcasted_iota(jnp.int32, (H, d), 0)).astype(f32)

    def gather_rows(scalars, base):
        rows = [enc_ref[pl.ds(base + b, 1), pl.ds(scalars[b], 1), :]
                for b in range(gs)]
        return jnp.concatenate(rows, axis=0)[:, 0, :]

    sels_ref[...] = jnp.zeros(sels_ref.shape, jnp.int32)

    st = []
    for gi in range(groups):
        base = gi * gs
        enc_g = enc_ref[pl.ds(base, gs), :, :]
        k_g = lax.dot_general(enc_g, wk_ref[...], (((2,), (0,)), ((), ())),
                              precision=df, preferred_element_type=f32)
        v_g = lax.dot_general(enc_g, wv_ref[...], (((2,), (0,)), ((), ())),
                              precision=df, preferred_element_type=f32)
        enc_t = jnp.swapaxes(enc_g, 1, 2).reshape(gs * d, p)
        k_t = jnp.swapaxes(k_g, 1, 2).reshape(gs * d, p)
        cur = [last_ref[0, 0, base + b] for b in range(gs)]
        g0 = gather_rows(cur, base)
        qf = jnp.dot(g0, wqf_ref[...], precision=df,
                     preferred_element_type=f32)
        ninf_g = ninf_ref[pl.ds(base, gs), 0, :]
        st.append(dict(base=base, enc_t=enc_t, k_t=k_t, v=v_g, qf=qf,
                       cur=cur, ninf=ninf_g))

    for t in range(n_steps):
        for s in st:
            g = gather_rows(s["cur"], s["base"])
            q = s["qf"] + jnp.dot(g, wql_ref[...], precision=df,
                                  preferred_element_type=f32)
            qh = (q[:, None, :] * gm[None]) * (1.0 / (dh ** 0.5))
            qhblk = (qh[:, :, None, :] * eye_g[:, None, :, None]
                     ).reshape(gs * H, gs * d)
            logits = jnp.dot(qhblk, s["k_t"], precision=df,
                             preferred_element_type=f32).reshape(gs, H, p)
            logits = logits + s["ninf"][:, None, :]
            m = jnp.max(logits, axis=2, keepdims=True)
            e = jnp.exp(logits - m)
            aw = e / jnp.sum(e, axis=2, keepdims=True)
            mh_g = lax.dot_general(aw, s["v"], (((2,), (1,)), ((0,), (0,))),
                                   precision=df, preferred_element_type=f32)
            mh = jnp.sum(mh_g * gm[None], axis=1)
            mo = jnp.dot(mh, wo_ref[...], precision=df,
                         preferred_element_type=f32)
            moblk = (mo[:, None, :] * eye_g[:, :, None]).reshape(gs, gs * d)
            sc = jnp.dot(moblk, s["enc_t"], precision=df,
                         preferred_element_type=f32)
            sc = sc + s["ninf"]
            m2 = jnp.max(sc, axis=1, keepdims=True)
            idx = jnp.min(jnp.where(sc == m2, iota_p, p), axis=1,
                          keepdims=True)
            sels_ref[pl.ds(s["base"], gs), 0, t:t + 1] = idx
            s["ninf"] = jnp.where(iota_p == idx, NEG, s["ninf"])
            s["cur"] = [jnp.min(idx[b]) for b in range(gs)]


def _decode(encoded_nodes, ninf0, last2d, Wq_first, Wq_last, Wk, Wv, Wo,
            n_steps, cb):
    b, p, d = encoded_nodes.shape
    grid = b // cb
    wspec = pl.BlockSpec((d, d), lambda i: (0, 0))
    return pl.pallas_call(
        functools.partial(_decode_body, n_steps=n_steps, groups=1),
        grid=(grid,),
        in_specs=[
            pl.BlockSpec((cb, p, d), lambda i: (i, 0, 0)),
            pl.BlockSpec((cb, 1, p), lambda i: (i, 0, 0)),
            pl.BlockSpec((1, 1, cb), lambda i: (i, 0, 0),
                         memory_space=pltpu.SMEM),
            wspec, wspec, wspec, wspec, wspec,
        ],
        out_specs=pl.BlockSpec((cb, 1, 128), lambda i: (i, 0, 0)),
        out_shape=jax.ShapeDtypeStruct((b, 1, 128), jnp.int32),
    )(encoded_nodes, ninf0[:, None, :], last2d.reshape(grid, 1, cb),
      Wq_first, Wq_last, Wk, Wv, Wo)


def kernel(problems, encoded_nodes, prefix, Wq_first, Wq_last, Wk, Wv, Wo):
    b, p, d = encoded_nodes.shape
    kp = prefix.shape[1]
    n_steps = p - kp

    ninf0 = _sc_mask(prefix, p)

    last2d = prefix[:, -1]
    sels = _decode(encoded_nodes, ninf0, last2d, Wq_first, Wq_last, Wk, Wv, Wo,
                   n_steps, cb=8)
    return jnp.concatenate([prefix, sels[:, 0, :n_steps]], axis=1)
